# baseline (device time: 57703 ns/iter reference)
import jax
import jax.numpy as jnp
from jax import lax
from jax.experimental import pallas as pl
from jax.experimental.pallas import tpu as pltpu

N_DEV = 8
B = 2
SQ = 512
SKV = 512
H_PER = 8
DH = 64
D_MODEL = 768
D_HID = H_PER * DH

PCOL = D_MODEL // 3

ORDER = ((2, 1, 0), (1, 0, 2), (0, 2, 1))


def kernel(x, Wq, K_ext, V_ext, Wo):
    def body(x_ref, wq_ref, k_ref, v_ref, wo_ref, out_ref,
             send_ref, recv_ref, send_sems, recv_sems):
        my = lax.axis_index("i")
        lab = my ^ ((my >> 1) & 1)

        def pos_of(l):
            return l ^ ((l >> 1) & 1)

        qi = lax.broadcasted_iota(jnp.int32, (SQ, SKV), 0)
        ki = lax.broadcasted_iota(jnp.int32, (SQ, SKV), 1)
        mask = (jnp.abs(qi - ki) <= 128) | (ki < 32) | (qi < 32)
        neg = jnp.float32(-1e9)

        wq_s = wq_ref[:, pl.ds(my * D_HID, D_HID)] * 0.125

        def compute_half(b):
            qb = jnp.dot(x_ref[b], wq_s,
                         preferred_element_type=jnp.float32)
            acc = jnp.zeros((SQ, D_MODEL), jnp.float32)
            for h in range(H_PER):
                qh = qb[:, h * DH:(h + 1) * DH]
                kh = k_ref[b, :, h, :]
                s = lax.dot_general(
                    qh, kh, (((1,), (1,)), ((), ())),
                    preferred_element_type=jnp.float32)
                e = jnp.exp(jnp.where(mask, s, neg))
                v_aug = jnp.concatenate(
                    [v_ref[b, :, h, :], jnp.ones((SKV, 1), jnp.float32)],
                    axis=1)
                ctx_aug = jnp.dot(e, v_aug,
                                  preferred_element_type=jnp.float32)
                ctx = ctx_aug[:, :DH] / ctx_aug[:, DH:DH + 1]
                acc = acc + jnp.dot(
                    ctx, wo_ref[pl.ds(my * D_HID + h * DH, DH), :],
                    preferred_element_type=jnp.float32)
            send_ref[b] = acc.astype(jnp.bfloat16)

        def start_round(b, r):
            inflight = []
            for j in range(3):
                bit = ORDER[j][r]
                partner = pos_of(lab ^ (1 << bit))
                rdma = pltpu.make_async_remote_copy(
                    src_ref=send_ref.at[b, :, pl.ds(j * PCOL, PCOL)],
                    dst_ref=recv_ref.at[b, r, j],
                    send_sem=send_sems.at[b, r, j],
                    recv_sem=recv_sems.at[b, r, j],
                    device_id=(partner,),
                    device_id_type=pl.DeviceIdType.MESH,
                )
                rdma.start()
                inflight.append(rdma)
            return inflight

        def finish_round(b, r, inflight):
            for rdma in inflight:
                rdma.wait()
            for j in range(3):
                cols = pl.ds(j * PCOL, PCOL)
                send_ref[b, :, cols] = (
                    send_ref[b, :, cols] + recv_ref[b, r, j])
            if r == 2:
                out_ref[b] = send_ref[b].astype(jnp.float32)

        barrier = pltpu.get_barrier_semaphore()
        for bit in range(3):
            pl.semaphore_signal(
                barrier, inc=1,
                device_id=(pos_of(lab ^ (1 << bit)),),
                device_id_type=pl.DeviceIdType.MESH,
            )

        compute_half(0)
        pl.semaphore_wait(barrier, 3)
        fly0 = start_round(0, 0)
        compute_half(1)
        fly1 = start_round(1, 0)
        for r in range(3):
            finish_round(0, r, fly0)
            if r < 2:
                fly0 = start_round(0, r + 1)
            finish_round(1, r, fly1)
            if r < 2:
                fly1 = start_round(1, r + 1)

    return pl.pallas_call(
        body,
        out_shape=jax.ShapeDtypeStruct((B, SQ, D_MODEL), jnp.float32),
        in_specs=[pl.BlockSpec(memory_space=pltpu.VMEM)] * 5,
        out_specs=pl.BlockSpec(memory_space=pltpu.VMEM),
        scratch_shapes=[
            pltpu.VMEM((B, SQ, D_MODEL), jnp.bfloat16),
            pltpu.VMEM((B, 3, 3, SQ, PCOL), jnp.bfloat16),
            pltpu.SemaphoreType.DMA((B, 3, 3)),
            pltpu.SemaphoreType.DMA((B, 3, 3)),
        ],
        compiler_params=pltpu.CompilerParams(
            vmem_limit_bytes=100 * 1024 * 1024,
            collective_id=0,
        ),
    )(x, Wq, K_ext, V_ext, Wo)


# device time: 56652 ns/iter; 1.0186x vs baseline; 1.0186x over previous
import jax
import jax.numpy as jnp
from jax import lax
from jax.experimental import pallas as pl
from jax.experimental.pallas import tpu as pltpu

N_DEV = 8
B = 2
SQ = 512
SKV = 512
H_PER = 8
DH = 64
D_MODEL = 768
D_HID = H_PER * DH

PCOL = D_MODEL // 3

ORDER = ((2, 1, 0), (1, 0, 2), (0, 2, 1))


def kernel(x, Wq, K_ext, V_ext, Wo):
    def body(x_ref, wq_ref, k_ref, v_ref, wo_ref, out_ref,
             send_ref, recv_ref, send_sems, recv_sems):
        my = lax.axis_index("i")
        lab = my ^ ((my >> 1) & 1)

        def pos_of(l):
            return l ^ ((l >> 1) & 1)

        qi = lax.broadcasted_iota(jnp.int32, (SQ, SKV), 0)
        ki = lax.broadcasted_iota(jnp.int32, (SQ, SKV), 1)
        mask = (jnp.abs(qi - ki) <= 128) | (ki < 32) | (qi < 32)
        neg = jnp.float32(-1e9)

        wq_s = wq_ref[:, pl.ds(my * D_HID, D_HID)] * 0.125

        def compute_half(b):
            qb = jnp.dot(x_ref[b], wq_s,
                         preferred_element_type=jnp.float32)
            acc = jnp.zeros((SQ, D_MODEL), jnp.float32)
            for h in range(H_PER):
                qh = qb[:, h * DH:(h + 1) * DH]
                kh = k_ref[b, :, h, :]
                s = lax.dot_general(
                    qh, kh, (((1,), (1,)), ((), ())),
                    preferred_element_type=jnp.float32)
                e = jnp.exp(jnp.where(mask, s, neg))
                ctx = jnp.dot(e, v_ref[b, :, h, :],
                              preferred_element_type=jnp.float32)
                ctx = ctx / jnp.sum(e, axis=1, keepdims=True)
                acc = acc + jnp.dot(
                    ctx, wo_ref[pl.ds(my * D_HID + h * DH, DH), :],
                    preferred_element_type=jnp.float32)
            send_ref[b] = acc.astype(jnp.bfloat16)

        def start_round(b, r):
            inflight = []
            for j in range(3):
                bit = ORDER[j][r]
                partner = pos_of(lab ^ (1 << bit))
                rdma = pltpu.make_async_remote_copy(
                    src_ref=send_ref.at[b, :, pl.ds(j * PCOL, PCOL)],
                    dst_ref=recv_ref.at[b, r, j],
                    send_sem=send_sems.at[b, r, j],
                    recv_sem=recv_sems.at[b, r, j],
                    device_id=(partner,),
                    device_id_type=pl.DeviceIdType.MESH,
                )
                rdma.start()
                inflight.append(rdma)
            return inflight

        def finish_round(b, r, inflight):
            for rdma in inflight:
                rdma.wait()
            for j in range(3):
                cols = pl.ds(j * PCOL, PCOL)
                send_ref[b, :, cols] = (
                    send_ref[b, :, cols] + recv_ref[b, r, j])
            if r == 2:
                out_ref[b] = send_ref[b].astype(jnp.float32)

        barrier = pltpu.get_barrier_semaphore()
        for bit in range(3):
            pl.semaphore_signal(
                barrier, inc=1,
                device_id=(pos_of(lab ^ (1 << bit)),),
                device_id_type=pl.DeviceIdType.MESH,
            )

        compute_half(0)
        pl.semaphore_wait(barrier, 3)
        fly0 = start_round(0, 0)
        compute_half(1)
        fly1 = start_round(1, 0)
        for r in range(3):
            finish_round(0, r, fly0)
            if r < 2:
                fly0 = start_round(0, r + 1)
            finish_round(1, r, fly1)
            if r < 2:
                fly1 = start_round(1, r + 1)

    return pl.pallas_call(
        body,
        out_shape=jax.ShapeDtypeStruct((B, SQ, D_MODEL), jnp.float32),
        in_specs=[pl.BlockSpec(memory_space=pltpu.VMEM)] * 5,
        out_specs=pl.BlockSpec(memory_space=pltpu.VMEM),
        scratch_shapes=[
            pltpu.VMEM((B, SQ, D_MODEL), jnp.bfloat16),
            pltpu.VMEM((B, 3, 3, SQ, PCOL), jnp.bfloat16),
            pltpu.SemaphoreType.DMA((B, 3, 3)),
            pltpu.SemaphoreType.DMA((B, 3, 3)),
        ],
        compiler_params=pltpu.CompilerParams(
            vmem_limit_bytes=100 * 1024 * 1024,
            collective_id=0,
        ),
    )(x, Wq, K_ext, V_ext, Wo)
